# Initial kernel scaffold; baseline (speedup 1.0000x reference)
import functools

import jax
import jax.numpy as jnp
from jax import lax
from jax.experimental import pallas as pl
from jax.experimental.pallas import tpu as pltpu
from jax.experimental.pallas import tpu_sc as plsc

_NUM_CORES = 2
_NUM_SUBCORES = 16
_NUM_WORKERS = _NUM_CORES * _NUM_SUBCORES

_CHUNK = 80
_NBUF = 2


@functools.lru_cache(maxsize=None)
def _build_gather(n: int, dim: int):
    assert n % _NUM_WORKERS == 0
    b_per_w = n // _NUM_WORKERS
    assert b_per_w % _CHUNK == 0
    n_chunks = b_per_w // _CHUNK

    mesh = plsc.VectorSubcoreMesh(core_axis_name="c", subcore_axis_name="s")

    @functools.partial(
        pl.kernel,
        mesh=mesh,
        out_type=jax.ShapeDtypeStruct((n, dim), jnp.float32),
        scratch_types=[
            pltpu.VMEM((b_per_w,), jnp.int32),
            pltpu.VMEM((_NBUF, _CHUNK, dim), jnp.float32),
            pltpu.SemaphoreType.DMA((_NBUF,)),
        ],
    )
    def gather_kernel(idx_hbm, table_hbm, out_hbm, idx_v, rows_v, sems):
        wid = lax.axis_index("s") * _NUM_CORES + lax.axis_index("c")
        base = wid * b_per_w
        pltpu.sync_copy(idx_hbm.at[pl.ds(base, b_per_w)], idx_v)

        copies = [None] * n_chunks

        def start(c):
            b = c % _NBUF
            return pltpu.async_copy(
                table_hbm.at[idx_v.at[pl.ds(c * _CHUNK, _CHUNK)]],
                rows_v.at[b],
                sems.at[b],
            )

        for c in range(min(_NBUF, n_chunks)):
            copies[c] = start(c)
        for c in range(n_chunks):
            b = c % _NBUF
            copies[c].wait()
            pltpu.sync_copy(
                rows_v.at[b], out_hbm.at[pl.ds(base + c * _CHUNK, _CHUNK)]
            )
            nxt = c + _NBUF
            if nxt < n_chunks:
                copies[nxt] = start(nxt)

    return gather_kernel


@jax.jit
def kernel(vocab_id_list, table):
    batch, seq = vocab_id_list.shape
    _, dim = table.shape
    idx = vocab_id_list.reshape(-1)
    out = _build_gather(batch * seq, dim)(idx, table)
    return out.reshape(batch, seq, dim)


# speedup vs baseline: 2.2896x; 2.2896x over previous
import functools

import jax
import jax.numpy as jnp
from jax import lax
from jax.experimental import pallas as pl
from jax.experimental.pallas import tpu as pltpu
from jax.experimental.pallas import tpu_sc as plsc

_NC = 2
_NS = 16
_NW = _NC * _NS

VOCAB = 100000
DIM = 300
BATCH = 1024
SEQ = 50
HEAD = 256
TAIL = DIM - HEAD
DIMP = 384
B_PER_W = BATCH // _NW
_NBUF = 2

mesh = plsc.VectorSubcoreMesh(core_axis_name="c", subcore_axis_name="s")


@functools.partial(
    pl.kernel,
    mesh=mesh,
    out_type=jax.ShapeDtypeStruct((BATCH, SEQ, DIMP), jnp.float32),
    scratch_types=[
        pltpu.VMEM((B_PER_W, SEQ), jnp.int32),
        pltpu.VMEM((_NBUF, SEQ, DIMP), jnp.float32),
        pltpu.SemaphoreType.DMA((_NBUF,)),
    ],
)
def _gather_sc(idx_hbm, table_hbm, tail_hbm, out_hbm, idx_v, rows_v, sems):
    wid = lax.axis_index("s") * _NC + lax.axis_index("c")
    b0 = wid * B_PER_W
    pltpu.sync_copy(idx_hbm.at[pl.ds(b0, B_PER_W)], idx_v)

    def start(c):
        buf = c % _NBUF
        return (
            pltpu.async_copy(
                table_hbm.at[idx_v.at[c], pl.ds(0, 128)],
                rows_v.at[buf, :, pl.ds(0, 128)],
                sems.at[buf],
            ),
            pltpu.async_copy(
                table_hbm.at[idx_v.at[c], pl.ds(128, 128)],
                rows_v.at[buf, :, pl.ds(128, 128)],
                sems.at[buf],
            ),
            pltpu.async_copy(
                tail_hbm.at[idx_v.at[c]],
                rows_v.at[buf, :, pl.ds(256, 128)],
                sems.at[buf],
            ),
        )

    copies = [None] * B_PER_W
    for c in range(min(_NBUF, B_PER_W)):
        copies[c] = start(c)
    for c in range(B_PER_W):
        buf = c % _NBUF
        for cp in copies[c]:
            cp.wait()
        pltpu.sync_copy(rows_v.at[buf], out_hbm.at[b0 + c])
        nxt = c + _NBUF
        if nxt < B_PER_W:
            copies[nxt] = start(nxt)


@jax.jit
def kernel(vocab_id_list, table):
    tail = jnp.pad(table[:, HEAD:], ((0, 0), (0, 128 - TAIL)))
    out = _gather_sc(vocab_id_list, table, tail)
    return out[:, :, :DIM]


# speedup vs baseline: 2.3028x; 1.0058x over previous
import functools

import jax
import jax.numpy as jnp
from jax import lax
from jax.experimental import pallas as pl
from jax.experimental.pallas import tpu as pltpu
from jax.experimental.pallas import tpu_sc as plsc

_NC = 2
_NS = 16
_NW = _NC * _NS

VOCAB = 100000
DIM = 300
BATCH = 1024
SEQ = 50
HEAD = 256
TAIL = DIM - HEAD
SHIFT = DIM - 128
B_PER_W = BATCH // _NW
_NBUF = 2

mesh = plsc.VectorSubcoreMesh(core_axis_name="c", subcore_axis_name="s")


@functools.partial(
    pl.kernel,
    mesh=mesh,
    out_type=(
        jax.ShapeDtypeStruct((BATCH, SEQ, DIM), jnp.float32),
        jax.ShapeDtypeStruct((BATCH, SEQ, 128), jnp.float32),
    ),
    scratch_types=[
        pltpu.VMEM((B_PER_W, SEQ), jnp.int32),
        pltpu.VMEM((_NBUF, SEQ, 128), jnp.float32),
        pltpu.VMEM((_NBUF, SEQ, 128), jnp.float32),
        pltpu.VMEM((_NBUF, SEQ, 128), jnp.float32),
        pltpu.SemaphoreType.DMA((_NBUF,)),
    ],
)
def _gather_sc(idx_hbm, table_hbm, tailsrc_hbm, out_hbm, tail_hbm,
               idx_v, buf_a, buf_b, buf_t, sems):
    wid = lax.axis_index("s") * _NC + lax.axis_index("c")
    b0 = wid * B_PER_W
    pltpu.sync_copy(idx_hbm.at[pl.ds(b0, B_PER_W)], idx_v)

    def start(c, buf):
        pltpu.async_copy(
            table_hbm.at[idx_v.at[c], pl.ds(0, 128)], buf_a.at[buf],
            sems.at[buf],
        )
        pltpu.async_copy(
            table_hbm.at[idx_v.at[c], pl.ds(128, 128)], buf_b.at[buf],
            sems.at[buf],
        )
        pltpu.async_copy(
            tailsrc_hbm.at[idx_v.at[c]], buf_t.at[buf], sems.at[buf],
        )

    def wait(c, buf):
        pltpu.make_async_copy(
            table_hbm.at[idx_v.at[c], pl.ds(0, 128)], buf_a.at[buf],
            sems.at[buf],
        ).wait()
        pltpu.make_async_copy(
            table_hbm.at[idx_v.at[c], pl.ds(128, 128)], buf_b.at[buf],
            sems.at[buf],
        ).wait()
        pltpu.make_async_copy(
            tailsrc_hbm.at[idx_v.at[c]], buf_t.at[buf], sems.at[buf],
        ).wait()

    for b in range(_NBUF):
        start(b, b)

    @pl.loop(0, B_PER_W, step=_NBUF)
    def _chunks(g):
        for b in range(_NBUF):
            c = g + b
            wait(c, b)
            pltpu.sync_copy(buf_a.at[b], out_hbm.at[b0 + c, :, pl.ds(0, 128)])
            pltpu.sync_copy(buf_b.at[b], out_hbm.at[b0 + c, :, pl.ds(128, 128)])
            pltpu.sync_copy(buf_t.at[b], tail_hbm.at[b0 + c])

            @pl.when(c + _NBUF < B_PER_W)
            def _():
                start(c + _NBUF, b)


@jax.jit
def kernel(vocab_id_list, table):
    tailsrc = lax.slice(table, (0, SHIFT), (VOCAB, DIM))
    out, tail = _gather_sc(vocab_id_list, table, tailsrc)
    tail44 = lax.slice(tail, (0, 0, 128 - TAIL), (BATCH, SEQ, 128))
    return lax.dynamic_update_slice(out, tail44, (0, 0, HEAD))
